# Initial kernel scaffold; baseline (speedup 1.0000x reference)
import functools

import jax
import jax.numpy as jnp
from jax import lax
from jax.experimental import pallas as pl
from jax.experimental.pallas import tpu as pltpu
from jax.experimental.pallas import tpu_sc as plsc

_C = 16
_N = 2_000_000
_NW = 32
_L = 16
_B = 256
_NB = 245
_PER_W = _B * _NB
_PAD_N = _NW * _PER_W
_V = 128 * 128 * 128


def _sc_sampler():
    mesh = plsc.VectorSubcoreMesh(core_axis_name="c", subcore_axis_name="s")

    @functools.partial(
        pl.kernel,
        mesh=mesh,
        out_type=jax.ShapeDtypeStruct((_PAD_N, _C), jnp.float32),
        scratch_types=[
            pltpu.VMEM((3, _B), jnp.float32),
            pltpu.VMEM((8, _B), jnp.int32),
            pltpu.VMEM((8, _B), jnp.float32),
            pltpu.VMEM((8, _B, _C), jnp.float32),
            pltpu.VMEM((_B, _C), jnp.float32),
            pltpu.SemaphoreType.DMA,
        ],
    )
    def sampler(tab_hbm, crd_hbm, out_hbm, crd_v, idx_v, w_v, rows_v, out_v,
                sem):
        wid = lax.axis_index("s") * 2 + lax.axis_index("c")
        base = wid * _PER_W
        iota = lax.iota(jnp.int32, _L)

        def batch(b, carry):
            off = base + b * _B
            for d in range(3):
                pltpu.sync_copy(crd_hbm.at[d, pl.ds(off, _B)], crd_v.at[d])

            def phase_a(g, c2):
                s = g * _L
                sl = pl.ds(s, _L)
                f = []
                for d in range(3):
                    c = crd_v[d, sl]
                    f.append(jnp.clip(c * 63.5 + 63.5, 0.0, 127.0))
                fx, fy, fz = f
                x0 = fx.astype(jnp.int32)
                y0 = fy.astype(jnp.int32)
                z0 = fz.astype(jnp.int32)
                wx1 = fx - x0.astype(jnp.float32)
                wy1 = fy - y0.astype(jnp.float32)
                wz1 = fz - z0.astype(jnp.float32)
                wx0 = 1.0 - wx1
                wy0 = 1.0 - wy1
                wz0 = 1.0 - wz1
                x1 = jnp.minimum(x0 + 1, 127)
                y1 = jnp.minimum(y0 + 1, 127)
                z1 = jnp.minimum(z0 + 1, 127)
                zs0 = z0 << 14
                zs1 = z1 << 14
                ys0 = y0 << 7
                ys1 = y1 << 7
                b00 = zs0 + ys0
                b01 = zs0 + ys1
                b10 = zs1 + ys0
                b11 = zs1 + ys1
                idx_v[0, sl] = b00 + x0
                idx_v[1, sl] = b00 + x1
                idx_v[2, sl] = b01 + x0
                idx_v[3, sl] = b01 + x1
                idx_v[4, sl] = b10 + x0
                idx_v[5, sl] = b10 + x1
                idx_v[6, sl] = b11 + x0
                idx_v[7, sl] = b11 + x1
                w00 = wz0 * wy0
                w01 = wz0 * wy1
                w10 = wz1 * wy0
                w11 = wz1 * wy1
                w_v[0, sl] = w00 * wx0
                w_v[1, sl] = w00 * wx1
                w_v[2, sl] = w01 * wx0
                w_v[3, sl] = w01 * wx1
                w_v[4, sl] = w10 * wx0
                w_v[5, sl] = w10 * wx1
                w_v[6, sl] = w11 * wx0
                w_v[7, sl] = w11 * wx1
                return c2

            lax.fori_loop(0, _B // _L, phase_a, 0)

            copies = []
            for k in range(8):
                for j in range(_B // 128):
                    jsl = pl.ds(j * 128, 128)
                    copies.append(
                        pltpu.async_copy(
                            tab_hbm.at[idx_v.at[k, jsl]],
                            rows_v.at[k, jsl],
                            sem,
                        ))
            for cp in copies:
                cp.wait()

            def phase_b(g, c2):
                s = g * _L
                pvec = s + iota
                wks = [w_v[k, pl.ds(s, _L)] for k in range(8)]
                for c in range(_C):
                    cvec = jnp.full((_L,), c, jnp.int32)
                    acc = None
                    for k in range(8):
                        kvec = jnp.full((_L,), k, jnp.int32)
                        gk = plsc.load_gather(rows_v, [kvec, pvec, cvec])
                        t = gk * wks[k]
                        acc = t if acc is None else acc + t
                    plsc.store_scatter(out_v, [pvec, cvec], acc)
                return c2

            lax.fori_loop(0, _B // _L, phase_b, 0)

            pltpu.sync_copy(out_v, out_hbm.at[pl.ds(off, _B)])
            return carry

        lax.fori_loop(0, _NB, batch, 0)

    return sampler


def kernel(input, param):
    tab = input.transpose(1, 2, 3, 0).reshape(_V, _C)
    crd = jnp.pad(param.transpose(1, 0), ((0, 0), (0, _PAD_N - _N)))
    out = _sc_sampler()(tab, crd)
    return out[:_N]


# speedup vs baseline: 3.3962x; 3.3962x over previous
import functools

import jax
import jax.numpy as jnp
from jax import lax
from jax.experimental import pallas as pl
from jax.experimental.pallas import tpu as pltpu
from jax.experimental.pallas import tpu_sc as plsc

_C = 16
_N = 2_000_000
_NW = 32
_L = 16
_B = 256
_NB = 245
_PER_W = _B * _NB
_PAD_N = _NW * _PER_W
_V = 128 * 128 * 128


def _sc_sampler():
    mesh = plsc.VectorSubcoreMesh(core_axis_name="c", subcore_axis_name="s")

    @functools.partial(
        pl.kernel,
        mesh=mesh,
        compiler_params=pltpu.CompilerParams(use_tc_tiling_on_sc=False),
        out_type=jax.ShapeDtypeStruct((_PAD_N, _C), jnp.float32),
        scratch_types=[
            pltpu.VMEM((_B,), jnp.float32),
            pltpu.VMEM((_B,), jnp.float32),
            pltpu.VMEM((_B,), jnp.float32),
            pltpu.VMEM((8 * _B,), jnp.int32),
            pltpu.VMEM((8 * _B,), jnp.float32),
            pltpu.VMEM((8 * _B, _C), jnp.float32),
            pltpu.VMEM((_B, _C), jnp.float32),
            pltpu.SemaphoreType.DMA,
        ],
    )
    def sampler(tab_hbm, xs_hbm, ys_hbm, zs_hbm, out_hbm, xv, yv, zv,
                idx_v, w_v, rows_v, out_v, sem):
        wid = lax.axis_index("s") * 2 + lax.axis_index("c")
        base = wid * _PER_W
        iota = lax.iota(jnp.int32, _L)

        def batch(b, carry):
            off = base + b * _B
            pltpu.sync_copy(xs_hbm.at[pl.ds(off, _B)], xv)
            pltpu.sync_copy(ys_hbm.at[pl.ds(off, _B)], yv)
            pltpu.sync_copy(zs_hbm.at[pl.ds(off, _B)], zv)

            def phase_a(g, c2):
                s = g * _L
                sl = pl.ds(s, _L)
                fx = jnp.clip(xv[sl] * 63.5 + 63.5, 0.0, 127.0)
                fy = jnp.clip(yv[sl] * 63.5 + 63.5, 0.0, 127.0)
                fz = jnp.clip(zv[sl] * 63.5 + 63.5, 0.0, 127.0)
                x0 = fx.astype(jnp.int32)
                y0 = fy.astype(jnp.int32)
                z0 = fz.astype(jnp.int32)
                wx1 = fx - x0.astype(jnp.float32)
                wy1 = fy - y0.astype(jnp.float32)
                wz1 = fz - z0.astype(jnp.float32)
                wx0 = 1.0 - wx1
                wy0 = 1.0 - wy1
                wz0 = 1.0 - wz1
                x1 = jnp.minimum(x0 + 1, 127)
                y1 = jnp.minimum(y0 + 1, 127)
                z1 = jnp.minimum(z0 + 1, 127)
                zs0 = z0 << 14
                zs1 = z1 << 14
                ys0 = y0 << 7
                ys1 = y1 << 7
                b00 = zs0 + ys0
                b01 = zs0 + ys1
                b10 = zs1 + ys0
                b11 = zs1 + ys1
                bases = (b00, b00, b01, b01, b10, b10, b11, b11)
                xs_ = (x0, x1, x0, x1, x0, x1, x0, x1)
                for k in range(8):
                    idx_v[pl.ds(k * _B + s, _L)] = bases[k] + xs_[k]
                w00 = wz0 * wy0
                w01 = wz0 * wy1
                w10 = wz1 * wy0
                w11 = wz1 * wy1
                wzy = (w00, w00, w01, w01, w10, w10, w11, w11)
                wx = (wx0, wx1, wx0, wx1, wx0, wx1, wx0, wx1)
                for k in range(8):
                    w_v[pl.ds(k * _B + s, _L)] = wzy[k] * wx[k]
                return c2

            lax.fori_loop(0, _B // _L, phase_a, 0)

            copies = []
            for k in range(8):
                for j in range(_B // 128):
                    o = k * _B + j * 128
                    copies.append(
                        pltpu.async_copy(
                            tab_hbm.at[idx_v.at[pl.ds(o, 128)]],
                            rows_v.at[pl.ds(o, 128)],
                            sem,
                        ))
            for cp in copies:
                cp.wait()

            def phase_b(g, c2):
                s = g * _L
                wks = [w_v[pl.ds(k * _B + s, _L)] for k in range(8)]
                for p in range(_L):
                    pt = s + p
                    acc = rows_v[pt] * wks[0][p]
                    for k in range(1, 8):
                        acc = acc + rows_v[k * _B + pt] * wks[k][p]
                    out_v[pt] = acc
                return c2

            lax.fori_loop(0, _B // _L, phase_b, 0)

            pltpu.sync_copy(out_v, out_hbm.at[pl.ds(off, _B)])
            return carry

        lax.fori_loop(0, _NB, batch, 0)

    return sampler


def kernel(input, param):
    tab = input.transpose(1, 2, 3, 0).reshape(_V, _C)
    crd = jnp.pad(param.transpose(1, 0), ((0, 0), (0, _PAD_N - _N)))
    out = _sc_sampler()(tab, crd[0], crd[1], crd[2])
    return out[:_N]


# speedup vs baseline: 4.2057x; 1.2383x over previous
import functools

import jax
import jax.numpy as jnp
from jax import lax
from jax.experimental import pallas as pl
from jax.experimental.pallas import tpu as pltpu
from jax.experimental.pallas import tpu_sc as plsc

_C = 16
_N = 2_000_000
_NW = 32
_L = 16
_B = 256
_NB = 246
_PER_W = _B * _NB
_PAD_N = _NW * _PER_W
_V = 128 * 128 * 128
_CHUNKS = _B // 128


def _sc_sampler():
    mesh = plsc.VectorSubcoreMesh(core_axis_name="c", subcore_axis_name="s")

    @functools.partial(
        pl.kernel,
        mesh=mesh,
        compiler_params=pltpu.CompilerParams(use_tc_tiling_on_sc=False),
        out_type=jax.ShapeDtypeStruct((_PAD_N, _C), jnp.float32),
        scratch_types=[
            pltpu.VMEM((3 * _B,), jnp.float32),
            pltpu.VMEM((3 * _B,), jnp.float32),
            pltpu.VMEM((8 * _B,), jnp.int32),
            pltpu.VMEM((8 * _B,), jnp.int32),
            pltpu.VMEM((8 * _B,), jnp.float32),
            pltpu.VMEM((8 * _B,), jnp.float32),
            pltpu.VMEM((8 * _B, _C), jnp.float32),
            pltpu.VMEM((8 * _B, _C), jnp.float32),
            pltpu.VMEM((_B, _C), jnp.float32),
            pltpu.SemaphoreType.DMA,
            pltpu.SemaphoreType.DMA,
        ],
    )
    def sampler(tab_hbm, crd_hbm, out_hbm, cb0, cb1, ix0, ix1, wb0, wb1,
                rw0, rw1, out_v, sg0, sg1):
        wid = lax.axis_index("s") * 2 + lax.axis_index("c")
        base = wid * _PER_W
        gb_base = wid * _NB
        cbufs, ixs, wbs, rws, sgs = (cb0, cb1), (ix0, ix1), (wb0, wb1), \
            (rw0, rw1), (sg0, sg1)

        def prep(gb, par):
            cbuf, ixb, wbf, rwb, sem = \
                cbufs[par], ixs[par], wbs[par], rws[par], sgs[par]
            pltpu.sync_copy(crd_hbm.at[pl.ds(gb * 3 * _B, 3 * _B)], cbuf)

            def phase_a(g, c2):
                s = g * _L
                fx = jnp.clip(cbuf[pl.ds(s, _L)] * 63.5 + 63.5, 0.0, 127.0)
                fy = jnp.clip(cbuf[pl.ds(_B + s, _L)] * 63.5 + 63.5,
                              0.0, 127.0)
                fz = jnp.clip(cbuf[pl.ds(2 * _B + s, _L)] * 63.5 + 63.5,
                              0.0, 127.0)
                x0 = fx.astype(jnp.int32)
                y0 = fy.astype(jnp.int32)
                z0 = fz.astype(jnp.int32)
                wx1 = fx - x0.astype(jnp.float32)
                wy1 = fy - y0.astype(jnp.float32)
                wz1 = fz - z0.astype(jnp.float32)
                wx0 = 1.0 - wx1
                wy0 = 1.0 - wy1
                wz0 = 1.0 - wz1
                x1 = jnp.minimum(x0 + 1, 127)
                ys0 = y0 << 7
                ys1 = jnp.minimum(y0 + 1, 127) << 7
                zs0 = z0 << 14
                zs1 = jnp.minimum(z0 + 1, 127) << 14
                b00 = zs0 + ys0
                b01 = zs0 + ys1
                b10 = zs1 + ys0
                b11 = zs1 + ys1
                bases = (b00, b00, b01, b01, b10, b10, b11, b11)
                xks = (x0, x1, x0, x1, x0, x1, x0, x1)
                for k in range(8):
                    ixb[pl.ds(k * _B + s, _L)] = bases[k] + xks[k]
                w00 = wz0 * wy0
                w01 = wz0 * wy1
                w10 = wz1 * wy0
                w11 = wz1 * wy1
                wzy = (w00, w00, w01, w01, w10, w10, w11, w11)
                wxk = (wx0, wx1, wx0, wx1, wx0, wx1, wx0, wx1)
                for k in range(8):
                    wbf[pl.ds(k * _B + s, _L)] = wzy[k] * wxk[k]
                return c2

            lax.fori_loop(0, _B // _L, phase_a, 0)
            for j in range(8 * _CHUNKS):
                jsl = pl.ds(j * 128, 128)
                pltpu.async_copy(tab_hbm.at[ixb.at[jsl]], rwb.at[jsl], sem)

        def drain(par):
            pltpu.make_async_copy(tab_hbm.at[pl.ds(0, 8 * _B)], rws[par],
                                  sgs[par]).wait()

        def finish(b, par):
            wbf, rwb = wbs[par], rws[par]
            drain(par)

            def phase_b(g, c2):
                s = g * _L
                wks = [wbf[pl.ds(k * _B + s, _L)] for k in range(8)]
                for p in range(_L):
                    pt = s + p
                    acc = rwb[pt] * wks[0][p]
                    for k in range(1, 8):
                        acc = acc + rwb[k * _B + pt] * wks[k][p]
                    out_v[pt] = acc
                return c2

            lax.fori_loop(0, _B // _L, phase_b, 0)
            pltpu.sync_copy(out_v, out_hbm.at[pl.ds(base + b * _B, _B)])

        prep(gb_base, 0)

        def pipe(bb, carry):
            for par in (0, 1):
                b = 2 * bb + par
                prep(gb_base + b + 1, 1 - par)
                finish(b, par)
            return carry

        lax.fori_loop(0, _NB // 2, pipe, 0)
        drain(0)

    return sampler


def kernel(input, param):
    tab = input.transpose(1, 2, 3, 0).reshape(_V, _C)
    crd = jnp.pad(param.transpose(1, 0), ((0, 0), (0, _PAD_N - _N)))
    crd = crd.reshape(3, _NW * _NB, _B).transpose(1, 0, 2).reshape(-1)
    crd = jnp.pad(crd, (0, 3 * _B))
    out = _sc_sampler()(tab, crd)
    return out[:_N]


# speedup vs baseline: 5.0087x; 1.1909x over previous
import functools

import jax
import jax.numpy as jnp
from jax import lax
from jax.experimental import pallas as pl
from jax.experimental.pallas import tpu as pltpu
from jax.experimental.pallas import tpu_sc as plsc

_C = 16
_N = 2_000_000
_NW = 32
_L = 16
_B = 256
_NFB = 244
_NBLK = _NFB + 1
_PADP = _NW * _NBLK * _B
_V = 128 * 128 * 128
_FULL_PER_W = _NFB * _B


def _sc_sampler():
    mesh = plsc.VectorSubcoreMesh(core_axis_name="c", subcore_axis_name="s")

    @functools.partial(
        pl.kernel,
        mesh=mesh,
        compiler_params=pltpu.CompilerParams(use_tc_tiling_on_sc=False),
        out_type=jax.ShapeDtypeStruct((_N, _C), jnp.float32),
        scratch_types=[
            pltpu.VMEM((3 * _B,), jnp.float32),
            pltpu.VMEM((3 * _B,), jnp.float32),
            pltpu.VMEM((8 * _B,), jnp.int32),
            pltpu.VMEM((8 * _B,), jnp.int32),
            pltpu.VMEM((8 * _B,), jnp.float32),
            pltpu.VMEM((8 * _B,), jnp.float32),
            pltpu.VMEM((8 * _B, _C), jnp.float32),
            pltpu.VMEM((8 * _B, _C), jnp.float32),
            pltpu.VMEM((_B, _C), jnp.float32),
            pltpu.SemaphoreType.DMA,
            pltpu.SemaphoreType.DMA,
        ],
    )
    def sampler(tab_hbm, crd_hbm, out_hbm, cb0, cb1, ix0, ix1, wb0, wb1,
                rw0, rw1, out_v, sg0, sg1):
        wid = lax.axis_index("s") * 2 + lax.axis_index("c")
        gb_base = wid * _NBLK
        obase = wid * _FULL_PER_W + 32 * jnp.minimum(wid, 24) \
            + 48 * jnp.maximum(wid - 24, 0)
        tailg = jnp.where(wid < 24, 2, 3)
        cbufs, ixs, wbs, rws, sgs = (cb0, cb1), (ix0, ix1), (wb0, wb1), \
            (rw0, rw1), (sg0, sg1)

        def prep(gb, par):
            cbuf, ixb, wbf, rwb, sem = \
                cbufs[par], ixs[par], wbs[par], rws[par], sgs[par]
            pltpu.sync_copy(crd_hbm.at[pl.ds(gb * 3 * _B, 3 * _B)], cbuf)

            def phase_a(g, c2):
                s = g * _L
                fx = jnp.clip(cbuf[pl.ds(s, _L)] * 63.5 + 63.5, 0.0, 127.0)
                fy = jnp.clip(cbuf[pl.ds(_B + s, _L)] * 63.5 + 63.5,
                              0.0, 127.0)
                fz = jnp.clip(cbuf[pl.ds(2 * _B + s, _L)] * 63.5 + 63.5,
                              0.0, 127.0)
                x0 = fx.astype(jnp.int32)
                y0 = fy.astype(jnp.int32)
                z0 = fz.astype(jnp.int32)
                wx1 = fx - x0.astype(jnp.float32)
                wy1 = fy - y0.astype(jnp.float32)
                wz1 = fz - z0.astype(jnp.float32)
                wx0 = 1.0 - wx1
                wy0 = 1.0 - wy1
                wz0 = 1.0 - wz1
                x1 = jnp.minimum(x0 + 1, 127)
                ys0 = y0 << 7
                ys1 = jnp.minimum(y0 + 1, 127) << 7
                zs0 = z0 << 14
                zs1 = jnp.minimum(z0 + 1, 127) << 14
                b00 = zs0 + ys0
                b01 = zs0 + ys1
                b10 = zs1 + ys0
                b11 = zs1 + ys1
                bases = (b00, b00, b01, b01, b10, b10, b11, b11)
                xks = (x0, x1, x0, x1, x0, x1, x0, x1)
                for k in range(8):
                    ixb[pl.ds(k * _B + s, _L)] = bases[k] + xks[k]
                w00 = wz0 * wy0
                w01 = wz0 * wy1
                w10 = wz1 * wy0
                w11 = wz1 * wy1
                wzy = (w00, w00, w01, w01, w10, w10, w11, w11)
                wxk = (wx0, wx1, wx0, wx1, wx0, wx1, wx0, wx1)
                for k in range(8):
                    wbf[pl.ds(k * _B + s, _L)] = wzy[k] * wxk[k]
                return c2

            lax.fori_loop(0, _B // _L, phase_a, 0)
            for j in range(8 * (_B // 128)):
                jsl = pl.ds(j * 128, 128)
                pltpu.async_copy(tab_hbm.at[ixb.at[jsl]], rwb.at[jsl], sem)

        def drain(par):
            pltpu.make_async_copy(tab_hbm.at[pl.ds(0, 8 * _B)], rws[par],
                                  sgs[par]).wait()

        def combine_group(g, par):
            wbf, rwb = wbs[par], rws[par]
            s = g * _L
            wks = [wbf[pl.ds(k * _B + s, _L)] for k in range(8)]
            for p in range(_L):
                pt = s + p
                acc = rwb[pt] * wks[0][p]
                for k in range(1, 8):
                    acc = acc + rwb[k * _B + pt] * wks[k][p]
                out_v[pt] = acc

        def finish(b, par):
            drain(par)

            def phase_b(g, c2):
                combine_group(g, par)
                return c2

            lax.fori_loop(0, _B // _L, phase_b, 0)
            pltpu.sync_copy(out_v, out_hbm.at[pl.ds(obase + b * _B, _B)])

        prep(gb_base, 0)

        def pipe(bb, carry):
            for par in (0, 1):
                b = 2 * bb + par
                prep(gb_base + b + 1, 1 - par)
                finish(b, par)
            return carry

        lax.fori_loop(0, _NFB // 2, pipe, 0)
        drain(0)

        def tail(g, carry):
            combine_group(g, 0)
            pltpu.sync_copy(
                out_v.at[pl.ds(g * _L, _L)],
                out_hbm.at[pl.ds(obase + _FULL_PER_W + g * _L, _L)])
            return carry

        lax.fori_loop(0, tailg, tail, 0)

    return sampler


def _obase(w):
    return w * _FULL_PER_W + 32 * min(w, 24) + 48 * max(w - 24, 0)


def kernel(input, param):
    tab = input.transpose(1, 2, 3, 0).reshape(_V, _C)
    seg = _NBLK * _B
    pt = jnp.pad(param.transpose(1, 0),
                 ((0, 0), (0, _obase(_NW - 1) + seg - _N)))
    segs = jnp.stack(
        [lax.slice(pt, (0, _obase(w)), (3, _obase(w) + seg))
         for w in range(_NW)])
    crd = segs.reshape(_NW, 3, _NBLK, _B).transpose(0, 2, 1, 3).reshape(-1)
    return _sc_sampler()(tab, crd)
